# baseline (speedup 1.0000x reference)
import functools

import jax
import jax.numpy as jnp
from jax.experimental import pallas as pl
from jax.experimental.pallas import tpu as pltpu


def _bf(x):
    return x.astype(jnp.bfloat16)


def _bff(x):
    return x.astype(jnp.bfloat16).astype(jnp.float32)


def _fused_kernel(n_rows, c_rows,
                  x_ref, c_ref, w1_ref, b1_ref, w2_ref, b2_ref,
                  wc1_ref, bc1_ref, wfc_ref, bfc_ref, wc2_ref, bc2_ref,
                  wcomb_ref, bcomb_ref, wout_ref, bout_ref,
                  out_ref, acc_n, acc_c):
    b = pl.program_id(0)
    j = pl.program_id(1)
    nb = pl.num_programs(0)
    nj = pl.num_programs(1)

    @pl.when(jnp.logical_and(b == 0, j == 0))
    def _init():
        acc_n[...] = jnp.zeros_like(acc_n)
        acc_c[...] = jnp.zeros_like(acc_c)

    f32 = jnp.float32

    h = jnp.maximum(
        jnp.dot(_bf(x_ref[...]), w1_ref[...], preferred_element_type=f32)
        + b1_ref[...], 0.0)
    h = jnp.maximum(
        jnp.dot(_bf(h), w2_ref[...], preferred_element_type=f32)
        + b2_ref[...], 0.0)
    ns = jnp.sum(_bff(h), axis=0, keepdims=True)

    ch = jnp.maximum(
        jnp.dot(_bf(c_ref[...]), wc1_ref[...], preferred_element_type=f32)
        + bc1_ref[...], 0.0)
    cs = jnp.sum(_bff(ch), axis=0, keepdims=True)

    nbatch = acc_n.shape[0]
    row = jax.lax.broadcasted_iota(jnp.int32, (nbatch, 16), 0)
    sel = row == b
    acc_n[...] += jnp.where(sel, jnp.broadcast_to(ns, (nbatch, 16)), 0.0)
    acc_c[...] += jnp.where(sel, jnp.broadcast_to(cs, (nbatch, 16)), 0.0)

    @pl.when(jnp.logical_and(b == nb - 1, j == nj - 1))
    def _finish():
        node_avg = (jnp.sum(acc_n[...] * wfc_ref[...], axis=1, keepdims=True)
                    * (1.0 / n_rows) + bfc_ref[...])
        col_avg = (jnp.sum(acc_c[...] * wc2_ref[...], axis=1, keepdims=True)
                   * (1.0 / c_rows) + bc2_ref[...])
        z = jnp.maximum(
            _bff(node_avg) * wcomb_ref[0:1, :]
            + _bff(col_avg) * wcomb_ref[1:2, :]
            + bcomb_ref[...], 0.0)
        out_ref[...] = (jnp.sum(_bff(z) * wout_ref[...], axis=1, keepdims=True)
                        + bout_ref[...])


def kernel(node_features, col_features, edge_index, W1, b1, W2, b2, Wfc, bfc,
           Wc1, bc1, Wc2, bc2, Wcomb, bcomb, Wout, bout):
    del edge_index
    B, N, F = node_features.shape
    Bc, C, Fc = col_features.shape
    assert (B, F) == (Bc, Fc) and C == N and B == 4, "pipeline shapes"

    x2 = node_features.reshape(B * N, F)
    c2 = col_features.reshape(B * C, F)

    j_per_batch = 1
    for j in (5, 4, 8, 2, 10, 16):
        if N % j == 0 and N // j <= 2500:
            j_per_batch = j
            break
    R = N // j_per_batch

    H = W1.shape[1]
    b1r = b1.reshape(1, H)
    b2r = b2.reshape(1, H)
    bc1r = bc1.reshape(1, H)
    bfcr = bfc.reshape(1, 1)
    bc2r = bc2.reshape(1, 1)
    bcombr = bcomb.reshape(1, H)
    boutr = bout.reshape(1, 1)
    w1b, w2b, wc1b = _bf(W1), _bf(W2), _bf(Wc1)
    wfcr = _bff(Wfc).reshape(1, H)
    wc2r = _bff(Wc2).reshape(1, H)
    wcombr = _bff(Wcomb)
    woutr = _bff(Wout).reshape(1, H)

    row_spec = pl.BlockSpec((R, F), lambda b, j: (b * j_per_batch + j, 0))
    full = lambda arr: pl.BlockSpec(arr.shape, lambda b, j: (0,) * arr.ndim)

    out = pl.pallas_call(
        functools.partial(_fused_kernel, N, C),
        grid=(B, j_per_batch),
        in_specs=[
            row_spec, row_spec,
            full(w1b), full(b1r), full(w2b), full(b2r),
            full(wc1b), full(bc1r), full(wfcr), full(bfcr),
            full(wc2r), full(bc2r), full(wcombr), full(bcombr),
            full(woutr), full(boutr),
        ],
        out_specs=pl.BlockSpec((B, 1), lambda b, j: (0, 0)),
        out_shape=jax.ShapeDtypeStruct((B, 1), jnp.float32),
        scratch_shapes=[
            pltpu.VMEM((B, H), jnp.float32),
            pltpu.VMEM((B, H), jnp.float32),
        ],
        compiler_params=pltpu.CompilerParams(
            dimension_semantics=("arbitrary", "arbitrary"),
        ),
    )(x2, c2, w1b, b1r, w2b, b2r, wc1b, bc1r, wfcr, bfcr,
      wc2r, bc2r, wcombr, bcombr, woutr, boutr)
    return out
